# Initial kernel scaffold; baseline (speedup 1.0000x reference)
import jax
import jax.numpy as jnp
from jax.experimental import pallas as pl


def kernel(logits, edge_index, fc1_w, fc1_b, ln1_g, ln1_b, msg_w, msg_b, upd_w, upd_b, fc2_w, fc2_b, ln2_g, ln2_b, out_w, out_b, skip_w):
    raise NotImplementedError("write your pallas kernel here")


# speedup vs baseline: 9.3520x; 9.3520x over previous
import functools

import jax
import jax.numpy as jnp
from jax import lax
from jax.experimental import pallas as pl
from jax.experimental.pallas import tpu as pltpu
from jax.experimental.pallas import tpu_sc as plsc

B = 8
L = 10000
E = 160000
H = 128

NC = 2
NS = 16
CHUNK = 128
CHUNKS_PER_TILE = 80
E_PAD = NS * CHUNKS_PER_TILE * CHUNK
L_PAD = 10240
ROWS_PER_TILE = 640
B_PER_CORE = B // NC

_RB = 1000
_GRID = (B * L) // _RB

_INV_SQRT2 = 0.7071067811865476


def _gelu(x):
    return 0.5 * x * (1.0 + lax.erf(x * _INV_SQRT2))


def _phase1_body(x_ref, w1_ref, b1_ref, g1_ref, bb1_ref, mw_ref, mb_ref,
                 out_ref):
    x = x_ref[...]
    h = x * w1_ref[...] + b1_ref[...]
    mu = jnp.mean(h, axis=-1, keepdims=True)
    var = jnp.mean((h - mu) ** 2, axis=-1, keepdims=True)
    h = (h - mu) * lax.rsqrt(var + 1e-5) * g1_ref[...] + bb1_ref[...]
    h = _gelu(h)
    out_ref[...] = (
        jnp.dot(h, mw_ref[...], preferred_element_type=jnp.float32)
        + mb_ref[...]
    )


def _phase1(x2d, fc1_w, fc1_b, ln1_g, ln1_b, msg_w, msg_b):
    full = lambda i: (0, 0)
    row = lambda i: (i, 0)
    return pl.pallas_call(
        _phase1_body,
        grid=(_GRID,),
        in_specs=[
            pl.BlockSpec((_RB, 1), row),
            pl.BlockSpec((1, H), full),
            pl.BlockSpec((1, H), full),
            pl.BlockSpec((1, H), full),
            pl.BlockSpec((1, H), full),
            pl.BlockSpec((H, H), full),
            pl.BlockSpec((1, H), full),
        ],
        out_specs=pl.BlockSpec((_RB, H), row),
        out_shape=jax.ShapeDtypeStruct((B * L, H), jnp.float32),
    )(x2d, fc1_w, fc1_b, ln1_g, ln1_b, msg_w, msg_b)


HH = H // 2


def _sc_body(msg_hbm, src_hbm, dst_hbm, zeros64_hbm, zeros16_hbm, ones16_hbm,
             agg_lo_hbm, agg_hi_hbm, deg_hbm,
             src_v, dst_v, gbuf_v, zeros64_v, zeros16_v, ones16_v,
             agg_sh, deg_sh):
    c = lax.axis_index("c")
    s = lax.axis_index("s")
    row0 = s * ROWS_PER_TILE
    n_last = L - (NS - 1) * ROWS_PER_TILE

    pltpu.sync_copy(dst_hbm.at[s], dst_v)
    pltpu.sync_copy(zeros64_hbm, zeros64_v)
    pltpu.sync_copy(zeros16_hbm, zeros16_v)
    pltpu.sync_copy(ones16_hbm, ones16_v)

    @pl.when(c == 0)
    def _deg():
        for j in range(5):
            pltpu.sync_copy(zeros16_v, deg_sh.at[pl.ds(row0 + j * CHUNK, CHUNK)])
        plsc.subcore_barrier()

        def deg_chunk(i, carry):
            pltpu.sync_copy(ones16_v, deg_sh.at[dst_v.at[i]], add=True)
            return carry
        lax.fori_loop(0, CHUNKS_PER_TILE, deg_chunk, 0)
        plsc.subcore_barrier()

        @pl.when(s < NS - 1)
        def _():
            pltpu.sync_copy(deg_sh.at[pl.ds(row0, ROWS_PER_TILE)],
                            deg_hbm.at[pl.ds(row0, ROWS_PER_TILE)])

        @pl.when(s == NS - 1)
        def _():
            pltpu.sync_copy(deg_sh.at[pl.ds(row0, n_last)],
                            deg_hbm.at[pl.ds(row0, n_last)])

    for bb in range(B_PER_CORE):
        for half in range(2):
            b = c * B_PER_CORE + bb
            out_hbm = agg_lo_hbm if half == 0 else agg_hi_hbm

            for j in range(5):
                pltpu.sync_copy(zeros64_v,
                                agg_sh.at[pl.ds(row0 + j * CHUNK, CHUNK)])
            pltpu.sync_copy(src_hbm.at[(b * 2 + half) * NS + s], src_v)
            plsc.subcore_barrier()

            def edge_chunk(i, carry):
                pltpu.sync_copy(msg_hbm.at[src_v.at[i]], gbuf_v)
                pltpu.sync_copy(gbuf_v, agg_sh.at[dst_v.at[i]], add=True)
                return carry
            lax.fori_loop(0, CHUNKS_PER_TILE, edge_chunk, 0)
            plsc.subcore_barrier()

            hbase = b * L + row0

            @pl.when(s < NS - 1)
            def _():
                pltpu.sync_copy(agg_sh.at[pl.ds(row0, ROWS_PER_TILE)],
                                out_hbm.at[pl.ds(hbase, ROWS_PER_TILE)])

            @pl.when(s == NS - 1)
            def _():
                pltpu.sync_copy(agg_sh.at[pl.ds(row0, n_last)],
                                out_hbm.at[pl.ds(hbase, n_last)])

            plsc.subcore_barrier()


def _phase2(msg64, src_abs, dst_tiles, zeros64, zeros16, ones16):
    mesh = plsc.VectorSubcoreMesh(core_axis_name="c", subcore_axis_name="s")
    f = pl.kernel(
        _sc_body,
        out_type=[
            jax.ShapeDtypeStruct((B * L, HH), jnp.float32),
            jax.ShapeDtypeStruct((B * L, HH), jnp.float32),
            jax.ShapeDtypeStruct((L, 16), jnp.float32),
        ],
        mesh=mesh,
        scratch_types=[
            pltpu.VMEM((CHUNKS_PER_TILE, CHUNK), jnp.int32),
            pltpu.VMEM((CHUNKS_PER_TILE, CHUNK), jnp.int32),
            pltpu.VMEM((CHUNK, HH), jnp.float32),
            pltpu.VMEM((CHUNK, HH), jnp.float32),
            pltpu.VMEM((CHUNK, 16), jnp.float32),
            pltpu.VMEM((CHUNK, 16), jnp.float32),
            pltpu.VMEM_SHARED((L_PAD, HH), jnp.float32),
            pltpu.VMEM_SHARED((L_PAD, 16), jnp.float32),
        ],
        compiler_params=pltpu.CompilerParams(use_tc_tiling_on_sc=False),
    )
    return f(msg64, src_abs, dst_tiles, zeros64, zeros16, ones16)


def _phase3_body(x_ref, agg_lo_ref, agg_hi_ref, deg_ref,
                 w1_ref, b1_ref, g1_ref, bb1_ref,
                 uwh_ref, uwa_ref, ub_ref,
                 f2w_ref, f2b_ref, g2_ref, bb2_ref,
                 ow_ref, ob_ref, sk_ref, out_ref):
    x = x_ref[...]
    h = x * w1_ref[...] + b1_ref[...]
    mu = jnp.mean(h, axis=-1, keepdims=True)
    var = jnp.mean((h - mu) ** 2, axis=-1, keepdims=True)
    h = (h - mu) * lax.rsqrt(var + 1e-5) * g1_ref[...] + bb1_ref[...]
    h = _gelu(h)

    a = (jnp.concatenate([agg_lo_ref[...], agg_hi_ref[...]], axis=-1)
         / jnp.maximum(deg_ref[...], 1.0))
    u = (jnp.dot(h, uwh_ref[...], preferred_element_type=jnp.float32)
         + jnp.dot(a, uwa_ref[...], preferred_element_type=jnp.float32)
         + ub_ref[...])
    u = _gelu(u)

    h2 = jnp.dot(u, f2w_ref[...], preferred_element_type=jnp.float32) + f2b_ref[...]
    mu2 = jnp.mean(h2, axis=-1, keepdims=True)
    var2 = jnp.mean((h2 - mu2) ** 2, axis=-1, keepdims=True)
    h2 = (h2 - mu2) * lax.rsqrt(var2 + 1e-5) * g2_ref[...] + bb2_ref[...]
    h2 = h2 + u
    h2 = _gelu(h2)

    refined = jnp.sum(h2 * ow_ref[...], axis=-1, keepdims=True) + ob_ref[...]
    alpha = jax.nn.sigmoid(sk_ref[...])
    out_ref[...] = alpha * refined + (1.0 - alpha) * x


def _phase3(x2d, agg_lo, agg_hi, deg2d, fc1_w, fc1_b, ln1_g, ln1_b,
            upd_wh, upd_wa, upd_b, fc2_w, fc2_b, ln2_g, ln2_b,
            out_w_row, out_b, skip_w2d):
    full = lambda i: (0, 0)
    row = lambda i: (i, 0)
    return pl.pallas_call(
        _phase3_body,
        grid=(_GRID,),
        in_specs=[
            pl.BlockSpec((_RB, 1), row),
            pl.BlockSpec((_RB, HH), row),
            pl.BlockSpec((_RB, HH), row),
            pl.BlockSpec((_RB, 1), lambda i: (i % (L // _RB), 0)),
            pl.BlockSpec((1, H), full),
            pl.BlockSpec((1, H), full),
            pl.BlockSpec((1, H), full),
            pl.BlockSpec((1, H), full),
            pl.BlockSpec((H, H), full),
            pl.BlockSpec((H, H), full),
            pl.BlockSpec((1, H), full),
            pl.BlockSpec((H, H), full),
            pl.BlockSpec((1, H), full),
            pl.BlockSpec((1, H), full),
            pl.BlockSpec((1, H), full),
            pl.BlockSpec((1, H), full),
            pl.BlockSpec((1, 1), full),
            pl.BlockSpec((1, 1), full),
        ],
        out_specs=pl.BlockSpec((_RB, 1), row),
        out_shape=jax.ShapeDtypeStruct((B * L, 1), jnp.float32),
    )(x2d, agg_lo, agg_hi, deg2d, fc1_w, fc1_b, ln1_g, ln1_b,
      upd_wh, upd_wa, upd_b, fc2_w, fc2_b, ln2_g, ln2_b,
      out_w_row, out_b, skip_w2d)


def kernel(logits, edge_index, fc1_w, fc1_b, ln1_g, ln1_b, msg_w, msg_b,
           upd_w, upd_b, fc2_w, fc2_b, ln2_g, ln2_b, out_w, out_b, skip_w):
    x2d = logits.reshape(B * L, 1)
    r = lambda v: v.reshape(1, H)

    msg_flat = _phase1(x2d, fc1_w.reshape(1, H), r(fc1_b), r(ln1_g), r(ln1_b),
                       msg_w, r(msg_b))

    src = edge_index[0]
    dst = edge_index[1]
    pad = E_PAD - E
    src_p = jnp.concatenate([src, jnp.zeros((pad,), jnp.int32)])
    dst_p = jnp.concatenate([dst, jnp.full((pad,), L, jnp.int32)])
    src_tiles = src_p.reshape(NS, CHUNKS_PER_TILE, CHUNK)
    boff = (jnp.arange(B, dtype=jnp.int32) * L)[:, None, None, None, None] * 2
    hoff = jnp.arange(2, dtype=jnp.int32)[None, :, None, None, None]
    src_abs = src_tiles[None, None] * 2 + boff + hoff
    src_abs = src_abs.reshape(B * 2 * NS, CHUNKS_PER_TILE, CHUNK)
    dst_tiles = dst_p.reshape(NS, CHUNKS_PER_TILE, CHUNK)

    msg64 = msg_flat.reshape(2 * B * L, HH)
    zeros64 = jnp.zeros((CHUNK, HH), jnp.float32)
    zeros16 = jnp.zeros((CHUNK, 16), jnp.float32)
    ones16 = jnp.ones((CHUNK, 16), jnp.float32)

    agg_lo, agg_hi, deg16 = _phase2(msg64, src_abs, dst_tiles,
                                    zeros64, zeros16, ones16)
    deg2d = deg16[:, :1]

    refined = _phase3(
        x2d, agg_lo, agg_hi, deg2d,
        fc1_w.reshape(1, H), r(fc1_b), r(ln1_g), r(ln1_b),
        upd_w[:H], upd_w[H:], r(upd_b),
        fc2_w, r(fc2_b), r(ln2_g), r(ln2_b),
        out_w.reshape(1, H), out_b.reshape(1, 1), skip_w.reshape(1, 1),
    )
    return refined.reshape(B, L)


# speedup vs baseline: 11.8771x; 1.2700x over previous
import functools

import jax
import jax.numpy as jnp
from jax import lax
from jax.experimental import pallas as pl
from jax.experimental.pallas import tpu as pltpu
from jax.experimental.pallas import tpu_sc as plsc

B = 8
L = 10000
E = 160000
H = 128

NC = 2
NS = 16
CHUNK = 128
CHUNKS_PER_TILE = 80
E_PAD = NS * CHUNKS_PER_TILE * CHUNK
L_PAD = 10240
ROWS_PER_TILE = 640
B_PER_CORE = B // NC

_RB = 1000
_GRID = (B * L) // _RB

_INV_SQRT2 = 0.7071067811865476


def _gelu(x):
    return 0.5 * x * (1.0 + lax.erf(x * _INV_SQRT2))


def _phase1_body(x_ref, w1_ref, b1_ref, g1_ref, bb1_ref, mw_ref, mb_ref,
                 out_ref):
    x = x_ref[...]
    h = x * w1_ref[...] + b1_ref[...]
    mu = jnp.mean(h, axis=-1, keepdims=True)
    var = jnp.mean((h - mu) ** 2, axis=-1, keepdims=True)
    h = (h - mu) * lax.rsqrt(var + 1e-5) * g1_ref[...] + bb1_ref[...]
    h = _gelu(h)
    out_ref[...] = (
        jnp.dot(h, mw_ref[...], preferred_element_type=jnp.float32)
        + mb_ref[...]
    )


def _phase1(x2d, fc1_w, fc1_b, ln1_g, ln1_b, msg_w, msg_b):
    full = lambda i: (0, 0)
    row = lambda i: (i, 0)
    return pl.pallas_call(
        _phase1_body,
        grid=(_GRID,),
        in_specs=[
            pl.BlockSpec((_RB, 1), row),
            pl.BlockSpec((1, H), full),
            pl.BlockSpec((1, H), full),
            pl.BlockSpec((1, H), full),
            pl.BlockSpec((1, H), full),
            pl.BlockSpec((H, H), full),
            pl.BlockSpec((1, H), full),
        ],
        out_specs=pl.BlockSpec((_RB, H), row),
        out_shape=jax.ShapeDtypeStruct((B * L, H), jnp.float32),
    )(x2d, fc1_w, fc1_b, ln1_g, ln1_b, msg_w, msg_b)


HH = H // 2


GRP = 2
N_GROUPS = CHUNKS_PER_TILE // GRP


def _sc_body(msg_hbm, src_hbm, dst_hbm, zeros64_hbm, zeros16_hbm, ones16_hbm,
             agg_lo_hbm, agg_hi_hbm, deg_hbm,
             src_v, dst_v, gbuf_v, gbuf2_v, zeros64_v, zeros16_v, ones16_v,
             gsem0, gsem1, agg_sh, deg_sh):
    c = lax.axis_index("c")
    s = lax.axis_index("s")
    row0 = s * ROWS_PER_TILE
    n_last = L - (NS - 1) * ROWS_PER_TILE

    pltpu.sync_copy(dst_hbm.at[s], dst_v)
    pltpu.sync_copy(zeros64_hbm, zeros64_v)
    pltpu.sync_copy(zeros16_hbm, zeros16_v)
    pltpu.sync_copy(ones16_hbm, ones16_v)

    @pl.when(c == 0)
    def _deg():
        for j in range(5):
            pltpu.sync_copy(zeros16_v, deg_sh.at[pl.ds(row0 + j * CHUNK, CHUNK)])
        plsc.subcore_barrier()

        def deg_chunk(i, carry):
            pltpu.sync_copy(ones16_v, deg_sh.at[dst_v.at[i]], add=True)
            return carry
        lax.fori_loop(0, CHUNKS_PER_TILE, deg_chunk, 0)
        plsc.subcore_barrier()

        @pl.when(s < NS - 1)
        def _():
            pltpu.sync_copy(deg_sh.at[pl.ds(row0, ROWS_PER_TILE)],
                            deg_hbm.at[pl.ds(row0, ROWS_PER_TILE)])

        @pl.when(s == NS - 1)
        def _():
            pltpu.sync_copy(deg_sh.at[pl.ds(row0, n_last)],
                            deg_hbm.at[pl.ds(row0, n_last)])

    for bb in range(B_PER_CORE):
        for half in range(2):
            b = c * B_PER_CORE + bb
            out_hbm = agg_lo_hbm if half == 0 else agg_hi_hbm

            for j in range(5):
                pltpu.sync_copy(zeros64_v,
                                agg_sh.at[pl.ds(row0 + j * CHUNK, CHUNK)])
            pltpu.sync_copy(src_hbm.at[(b * 2 + half) * NS + s], src_v)
            plsc.subcore_barrier()

            def _gissue(base, buf, sem):
                for t in range(GRP):
                    pltpu.async_copy(msg_hbm.at[src_v.at[base + t]],
                                     buf.at[pl.ds(t * CHUNK, CHUNK)], sem)

            def _gdrain(base, buf, sem):
                for t in range(GRP):
                    pltpu.make_async_copy(
                        msg_hbm.at[src_v.at[base + t]],
                        buf.at[pl.ds(t * CHUNK, CHUNK)], sem).wait()

            def _scat(base, buf):
                for t in range(GRP):
                    pltpu.sync_copy(buf.at[pl.ds(t * CHUNK, CHUNK)],
                                    agg_sh.at[dst_v.at[base + t]], add=True)

            _gissue(0, gbuf_v, gsem0)

            def edge_group(i, carry):
                ga = 2 * i * GRP
                gb = ga + GRP
                _gissue(gb, gbuf2_v, gsem1)
                _gdrain(ga, gbuf_v, gsem0)
                _scat(ga, gbuf_v)

                @pl.when(i < N_GROUPS // 2 - 1)
                def _():
                    _gissue(ga + 2 * GRP, gbuf_v, gsem0)

                _gdrain(gb, gbuf2_v, gsem1)
                _scat(gb, gbuf2_v)
                return carry
            lax.fori_loop(0, N_GROUPS // 2, edge_group, 0)
            plsc.subcore_barrier()

            hbase = b * L + row0

            @pl.when(s < NS - 1)
            def _():
                pltpu.sync_copy(agg_sh.at[pl.ds(row0, ROWS_PER_TILE)],
                                out_hbm.at[pl.ds(hbase, ROWS_PER_TILE)])

            @pl.when(s == NS - 1)
            def _():
                pltpu.sync_copy(agg_sh.at[pl.ds(row0, n_last)],
                                out_hbm.at[pl.ds(hbase, n_last)])

            plsc.subcore_barrier()


def _phase2(msg64, src_abs, dst_tiles, zeros64, zeros16, ones16):
    mesh = plsc.VectorSubcoreMesh(core_axis_name="c", subcore_axis_name="s")
    f = pl.kernel(
        _sc_body,
        out_type=[
            jax.ShapeDtypeStruct((B * L, HH), jnp.float32),
            jax.ShapeDtypeStruct((B * L, HH), jnp.float32),
            jax.ShapeDtypeStruct((L, 16), jnp.float32),
        ],
        mesh=mesh,
        scratch_types=[
            pltpu.VMEM((CHUNKS_PER_TILE, CHUNK), jnp.int32),
            pltpu.VMEM((CHUNKS_PER_TILE, CHUNK), jnp.int32),
            pltpu.VMEM((GRP * CHUNK, HH), jnp.float32),
            pltpu.VMEM((GRP * CHUNK, HH), jnp.float32),
            pltpu.VMEM((CHUNK, HH), jnp.float32),
            pltpu.VMEM((CHUNK, 16), jnp.float32),
            pltpu.VMEM((CHUNK, 16), jnp.float32),
            pltpu.SemaphoreType.DMA,
            pltpu.SemaphoreType.DMA,
            pltpu.VMEM_SHARED((L_PAD, HH), jnp.float32),
            pltpu.VMEM_SHARED((L_PAD, 16), jnp.float32),
        ],
        compiler_params=pltpu.CompilerParams(use_tc_tiling_on_sc=False),
    )
    return f(msg64, src_abs, dst_tiles, zeros64, zeros16, ones16)


def _phase3_body(x_ref, agg_lo_ref, agg_hi_ref, deg_ref,
                 w1_ref, b1_ref, g1_ref, bb1_ref,
                 uwh_ref, uwa_ref, ub_ref,
                 f2w_ref, f2b_ref, g2_ref, bb2_ref,
                 ow_ref, ob_ref, sk_ref, out_ref):
    x = x_ref[...]
    h = x * w1_ref[...] + b1_ref[...]
    mu = jnp.mean(h, axis=-1, keepdims=True)
    var = jnp.mean((h - mu) ** 2, axis=-1, keepdims=True)
    h = (h - mu) * lax.rsqrt(var + 1e-5) * g1_ref[...] + bb1_ref[...]
    h = _gelu(h)

    a = (jnp.concatenate([agg_lo_ref[...], agg_hi_ref[...]], axis=-1)
         / jnp.maximum(deg_ref[...], 1.0))
    u = (jnp.dot(h, uwh_ref[...], preferred_element_type=jnp.float32)
         + jnp.dot(a, uwa_ref[...], preferred_element_type=jnp.float32)
         + ub_ref[...])
    u = _gelu(u)

    h2 = jnp.dot(u, f2w_ref[...], preferred_element_type=jnp.float32) + f2b_ref[...]
    mu2 = jnp.mean(h2, axis=-1, keepdims=True)
    var2 = jnp.mean((h2 - mu2) ** 2, axis=-1, keepdims=True)
    h2 = (h2 - mu2) * lax.rsqrt(var2 + 1e-5) * g2_ref[...] + bb2_ref[...]
    h2 = h2 + u
    h2 = _gelu(h2)

    refined = jnp.sum(h2 * ow_ref[...], axis=-1, keepdims=True) + ob_ref[...]
    alpha = jax.nn.sigmoid(sk_ref[...])
    out_ref[...] = alpha * refined + (1.0 - alpha) * x


def _phase3(x2d, agg_lo, agg_hi, deg2d, fc1_w, fc1_b, ln1_g, ln1_b,
            upd_wh, upd_wa, upd_b, fc2_w, fc2_b, ln2_g, ln2_b,
            out_w_row, out_b, skip_w2d):
    full = lambda i: (0, 0)
    row = lambda i: (i, 0)
    return pl.pallas_call(
        _phase3_body,
        grid=(_GRID,),
        in_specs=[
            pl.BlockSpec((_RB, 1), row),
            pl.BlockSpec((_RB, HH), row),
            pl.BlockSpec((_RB, HH), row),
            pl.BlockSpec((_RB, 1), lambda i: (i % (L // _RB), 0)),
            pl.BlockSpec((1, H), full),
            pl.BlockSpec((1, H), full),
            pl.BlockSpec((1, H), full),
            pl.BlockSpec((1, H), full),
            pl.BlockSpec((H, H), full),
            pl.BlockSpec((H, H), full),
            pl.BlockSpec((1, H), full),
            pl.BlockSpec((H, H), full),
            pl.BlockSpec((1, H), full),
            pl.BlockSpec((1, H), full),
            pl.BlockSpec((1, H), full),
            pl.BlockSpec((1, H), full),
            pl.BlockSpec((1, 1), full),
            pl.BlockSpec((1, 1), full),
        ],
        out_specs=pl.BlockSpec((_RB, 1), row),
        out_shape=jax.ShapeDtypeStruct((B * L, 1), jnp.float32),
    )(x2d, agg_lo, agg_hi, deg2d, fc1_w, fc1_b, ln1_g, ln1_b,
      upd_wh, upd_wa, upd_b, fc2_w, fc2_b, ln2_g, ln2_b,
      out_w_row, out_b, skip_w2d)


def kernel(logits, edge_index, fc1_w, fc1_b, ln1_g, ln1_b, msg_w, msg_b,
           upd_w, upd_b, fc2_w, fc2_b, ln2_g, ln2_b, out_w, out_b, skip_w):
    x2d = logits.reshape(B * L, 1)
    r = lambda v: v.reshape(1, H)

    msg_flat = _phase1(x2d, fc1_w.reshape(1, H), r(fc1_b), r(ln1_g), r(ln1_b),
                       msg_w, r(msg_b))

    src = edge_index[0]
    dst = edge_index[1]
    pad = E_PAD - E
    src_p = jnp.concatenate([src, jnp.zeros((pad,), jnp.int32)])
    dst_p = jnp.concatenate([dst, jnp.full((pad,), L, jnp.int32)])
    src_tiles = src_p.reshape(NS, CHUNKS_PER_TILE, CHUNK)
    boff = (jnp.arange(B, dtype=jnp.int32) * L)[:, None, None, None, None] * 2
    hoff = jnp.arange(2, dtype=jnp.int32)[None, :, None, None, None]
    src_abs = src_tiles[None, None] * 2 + boff + hoff
    src_abs = src_abs.reshape(B * 2 * NS, CHUNKS_PER_TILE, CHUNK)
    dst_tiles = dst_p.reshape(NS, CHUNKS_PER_TILE, CHUNK)

    msg64 = msg_flat.reshape(2 * B * L, HH)
    zeros64 = jnp.zeros((CHUNK, HH), jnp.float32)
    zeros16 = jnp.zeros((CHUNK, 16), jnp.float32)
    ones16 = jnp.ones((CHUNK, 16), jnp.float32)

    agg_lo, agg_hi, deg16 = _phase2(msg64, src_abs, dst_tiles,
                                    zeros64, zeros16, ones16)
    deg2d = deg16[:, :1]

    refined = _phase3(
        x2d, agg_lo, agg_hi, deg2d,
        fc1_w.reshape(1, H), r(fc1_b), r(ln1_g), r(ln1_b),
        upd_w[:H], upd_w[H:], r(upd_b),
        fc2_w, r(fc2_b), r(ln2_g), r(ln2_b),
        out_w.reshape(1, H), out_b.reshape(1, 1), skip_w.reshape(1, 1),
    )
    return refined.reshape(B, L)


# speedup vs baseline: 17.4276x; 1.4673x over previous
import functools

import jax
import jax.numpy as jnp
from jax import lax
from jax.experimental import pallas as pl
from jax.experimental.pallas import tpu as pltpu
from jax.experimental.pallas import tpu_sc as plsc

B = 8
L = 10000
E = 160000
H = 128

NC = 2
NS = 16
CHUNK = 128
CHUNKS_PER_TILE = 80
E_PAD = NS * CHUNKS_PER_TILE * CHUNK
L_PAD = 10240
ROWS_PER_TILE = 640
B_PER_CORE = B // NC

_RB = 1000
_GRID = (B * L) // _RB

_INV_SQRT2 = 0.7071067811865476


def _gelu(x):
    return 0.5 * x * (1.0 + lax.erf(x * _INV_SQRT2))


def _phase1_body(x_ref, w1_ref, b1_ref, g1_ref, bb1_ref, mw_ref, mb_ref,
                 out_ref):
    x = x_ref[...]
    h = x * w1_ref[...] + b1_ref[...]
    mu = jnp.mean(h, axis=-1, keepdims=True)
    var = jnp.mean((h - mu) ** 2, axis=-1, keepdims=True)
    h = (h - mu) * lax.rsqrt(var + 1e-5) * g1_ref[...] + bb1_ref[...]
    h = _gelu(h)
    out_ref[...] = (
        jnp.dot(h, mw_ref[...], preferred_element_type=jnp.float32)
        + mb_ref[...]
    ).astype(jnp.bfloat16)


def _phase1(x2d, fc1_w, fc1_b, ln1_g, ln1_b, msg_w, msg_b):
    full = lambda i: (0, 0)
    row = lambda i: (i, 0)
    return pl.pallas_call(
        _phase1_body,
        grid=(_GRID,),
        in_specs=[
            pl.BlockSpec((_RB, 1), row),
            pl.BlockSpec((1, H), full),
            pl.BlockSpec((1, H), full),
            pl.BlockSpec((1, H), full),
            pl.BlockSpec((1, H), full),
            pl.BlockSpec((H, H), full),
            pl.BlockSpec((1, H), full),
        ],
        out_specs=pl.BlockSpec((_RB, H), row),
        out_shape=jax.ShapeDtypeStruct((B * L, H), jnp.bfloat16),
    )(x2d, fc1_w, fc1_b, ln1_g, ln1_b, msg_w, msg_b)


HH = H // 2


GRP = 2
N_GROUPS = CHUNKS_PER_TILE // GRP


def _sc_body(msg_hbm, src_hbm, dst_hbm, zerosh_hbm, zeros16_hbm, ones16_hbm,
             agg_hbm, deg_hbm,
             src_v, dst_v, gbuf_v, gbuf2_v, zerosh_v, zeros16_v, ones16_v,
             gsem0, gsem1, agg_sh, deg_sh):
    c = lax.axis_index("c")
    s = lax.axis_index("s")
    row0 = s * ROWS_PER_TILE
    n_last = L - (NS - 1) * ROWS_PER_TILE

    pltpu.sync_copy(dst_hbm.at[s], dst_v)
    pltpu.sync_copy(zerosh_hbm, zerosh_v)
    pltpu.sync_copy(zeros16_hbm, zeros16_v)
    pltpu.sync_copy(ones16_hbm, ones16_v)

    @pl.when(c == 0)
    def _deg():
        for j in range(5):
            pltpu.sync_copy(zeros16_v, deg_sh.at[pl.ds(row0 + j * CHUNK, CHUNK)])
        plsc.subcore_barrier()

        def deg_chunk(i, carry):
            pltpu.sync_copy(ones16_v, deg_sh.at[dst_v.at[i]], add=True)
            return carry
        lax.fori_loop(0, CHUNKS_PER_TILE, deg_chunk, 0)
        plsc.subcore_barrier()

        @pl.when(s < NS - 1)
        def _():
            pltpu.sync_copy(deg_sh.at[pl.ds(row0, ROWS_PER_TILE)],
                            deg_hbm.at[pl.ds(row0, ROWS_PER_TILE)])

        @pl.when(s == NS - 1)
        def _():
            pltpu.sync_copy(deg_sh.at[pl.ds(row0, n_last)],
                            deg_hbm.at[pl.ds(row0, n_last)])

    for bb in range(B_PER_CORE):
        if True:
            b = c * B_PER_CORE + bb
            out_hbm = agg_hbm

            for j in range(5):
                pltpu.sync_copy(zerosh_v,
                                agg_sh.at[pl.ds(row0 + j * CHUNK, CHUNK)])
            pltpu.sync_copy(src_hbm.at[b * NS + s], src_v)
            plsc.subcore_barrier()

            def _gissue(base, buf, sem):
                for t in range(GRP):
                    pltpu.async_copy(msg_hbm.at[src_v.at[base + t]],
                                     buf.at[pl.ds(t * CHUNK, CHUNK)], sem)

            def _gdrain(base, buf, sem):
                for t in range(GRP):
                    pltpu.make_async_copy(
                        msg_hbm.at[src_v.at[base + t]],
                        buf.at[pl.ds(t * CHUNK, CHUNK)], sem).wait()

            def _scat(base, buf):
                for t in range(GRP):
                    pltpu.sync_copy(buf.at[pl.ds(t * CHUNK, CHUNK)],
                                    agg_sh.at[dst_v.at[base + t]], add=True)

            _gissue(0, gbuf_v, gsem0)

            def edge_group(i, carry):
                ga = 2 * i * GRP
                gb = ga + GRP
                _gissue(gb, gbuf2_v, gsem1)
                _gdrain(ga, gbuf_v, gsem0)
                _scat(ga, gbuf_v)

                @pl.when(i < N_GROUPS // 2 - 1)
                def _():
                    _gissue(ga + 2 * GRP, gbuf_v, gsem0)

                _gdrain(gb, gbuf2_v, gsem1)
                _scat(gb, gbuf2_v)
                return carry
            lax.fori_loop(0, N_GROUPS // 2, edge_group, 0)
            plsc.subcore_barrier()

            hbase = b * L + row0

            @pl.when(s < NS - 1)
            def _():
                pltpu.sync_copy(agg_sh.at[pl.ds(row0, ROWS_PER_TILE)],
                                out_hbm.at[pl.ds(hbase, ROWS_PER_TILE)])

            @pl.when(s == NS - 1)
            def _():
                pltpu.sync_copy(agg_sh.at[pl.ds(row0, n_last)],
                                out_hbm.at[pl.ds(hbase, n_last)])

            plsc.subcore_barrier()


def _phase2(msgbf, src_abs, dst_tiles, zerosh, zeros16, ones16):
    mesh = plsc.VectorSubcoreMesh(core_axis_name="c", subcore_axis_name="s")
    f = pl.kernel(
        _sc_body,
        out_type=[
            jax.ShapeDtypeStruct((B * L, H), jnp.bfloat16),
            jax.ShapeDtypeStruct((L, 16), jnp.float32),
        ],
        mesh=mesh,
        scratch_types=[
            pltpu.VMEM((CHUNKS_PER_TILE, CHUNK), jnp.int32),
            pltpu.VMEM((CHUNKS_PER_TILE, CHUNK), jnp.int32),
            pltpu.VMEM((GRP * CHUNK, H), jnp.bfloat16),
            pltpu.VMEM((GRP * CHUNK, H), jnp.bfloat16),
            pltpu.VMEM((CHUNK, H), jnp.bfloat16),
            pltpu.VMEM((CHUNK, 16), jnp.float32),
            pltpu.VMEM((CHUNK, 16), jnp.float32),
            pltpu.SemaphoreType.DMA,
            pltpu.SemaphoreType.DMA,
            pltpu.VMEM_SHARED((L_PAD, H), jnp.bfloat16),
            pltpu.VMEM_SHARED((L_PAD, 16), jnp.float32),
        ],
        compiler_params=pltpu.CompilerParams(use_tc_tiling_on_sc=False),
    )
    return f(msgbf, src_abs, dst_tiles, zerosh, zeros16, ones16)


def _phase3_body(x_ref, agg_ref, deg_ref,
                 w1_ref, b1_ref, g1_ref, bb1_ref,
                 uwh_ref, uwa_ref, ub_ref,
                 f2w_ref, f2b_ref, g2_ref, bb2_ref,
                 ow_ref, ob_ref, sk_ref, out_ref):
    x = x_ref[...]
    h = x * w1_ref[...] + b1_ref[...]
    mu = jnp.mean(h, axis=-1, keepdims=True)
    var = jnp.mean((h - mu) ** 2, axis=-1, keepdims=True)
    h = (h - mu) * lax.rsqrt(var + 1e-5) * g1_ref[...] + bb1_ref[...]
    h = _gelu(h)

    a = agg_ref[...].astype(jnp.float32) / jnp.maximum(deg_ref[...], 1.0)
    u = (jnp.dot(h, uwh_ref[...], preferred_element_type=jnp.float32)
         + jnp.dot(a, uwa_ref[...], preferred_element_type=jnp.float32)
         + ub_ref[...])
    u = _gelu(u)

    h2 = jnp.dot(u, f2w_ref[...], preferred_element_type=jnp.float32) + f2b_ref[...]
    mu2 = jnp.mean(h2, axis=-1, keepdims=True)
    var2 = jnp.mean((h2 - mu2) ** 2, axis=-1, keepdims=True)
    h2 = (h2 - mu2) * lax.rsqrt(var2 + 1e-5) * g2_ref[...] + bb2_ref[...]
    h2 = h2 + u
    h2 = _gelu(h2)

    refined = jnp.sum(h2 * ow_ref[...], axis=-1, keepdims=True) + ob_ref[...]
    alpha = jax.nn.sigmoid(sk_ref[...])
    out_ref[...] = alpha * refined + (1.0 - alpha) * x


def _phase3(x2d, agg, deg2d, fc1_w, fc1_b, ln1_g, ln1_b,
            upd_wh, upd_wa, upd_b, fc2_w, fc2_b, ln2_g, ln2_b,
            out_w_row, out_b, skip_w2d):
    full = lambda i: (0, 0)
    row = lambda i: (i, 0)
    return pl.pallas_call(
        _phase3_body,
        grid=(_GRID,),
        in_specs=[
            pl.BlockSpec((_RB, 1), row),
            pl.BlockSpec((_RB, H), row),
            pl.BlockSpec((_RB, 1), lambda i: (i % (L // _RB), 0)),
            pl.BlockSpec((1, H), full),
            pl.BlockSpec((1, H), full),
            pl.BlockSpec((1, H), full),
            pl.BlockSpec((1, H), full),
            pl.BlockSpec((H, H), full),
            pl.BlockSpec((H, H), full),
            pl.BlockSpec((1, H), full),
            pl.BlockSpec((H, H), full),
            pl.BlockSpec((1, H), full),
            pl.BlockSpec((1, H), full),
            pl.BlockSpec((1, H), full),
            pl.BlockSpec((1, H), full),
            pl.BlockSpec((1, 1), full),
            pl.BlockSpec((1, 1), full),
        ],
        out_specs=pl.BlockSpec((_RB, 1), row),
        out_shape=jax.ShapeDtypeStruct((B * L, 1), jnp.float32),
    )(x2d, agg, deg2d, fc1_w, fc1_b, ln1_g, ln1_b,
      upd_wh, upd_wa, upd_b, fc2_w, fc2_b, ln2_g, ln2_b,
      out_w_row, out_b, skip_w2d)


def kernel(logits, edge_index, fc1_w, fc1_b, ln1_g, ln1_b, msg_w, msg_b,
           upd_w, upd_b, fc2_w, fc2_b, ln2_g, ln2_b, out_w, out_b, skip_w):
    x2d = logits.reshape(B * L, 1)
    r = lambda v: v.reshape(1, H)

    msg_flat = _phase1(x2d, fc1_w.reshape(1, H), r(fc1_b), r(ln1_g), r(ln1_b),
                       msg_w, r(msg_b))

    src = edge_index[0]
    dst = edge_index[1]
    pad = E_PAD - E
    src_p = jnp.concatenate([src, jnp.zeros((pad,), jnp.int32)])
    dst_p = jnp.concatenate([dst, jnp.full((pad,), L, jnp.int32)])
    src_tiles = src_p.reshape(NS, CHUNKS_PER_TILE, CHUNK)
    src_abs = (src_tiles[None] +
               (jnp.arange(B, dtype=jnp.int32) * L)[:, None, None, None])
    src_abs = src_abs.reshape(B * NS, CHUNKS_PER_TILE, CHUNK)
    dst_tiles = dst_p.reshape(NS, CHUNKS_PER_TILE, CHUNK)

    zerosh = jnp.zeros((CHUNK, H), jnp.bfloat16)
    zeros16 = jnp.zeros((CHUNK, 16), jnp.float32)
    ones16 = jnp.ones((CHUNK, 16), jnp.float32)

    agg, deg16 = _phase2(msg_flat, src_abs, dst_tiles,
                         zerosh, zeros16, ones16)
    deg2d = deg16[:, :1]

    refined = _phase3(
        x2d, agg, deg2d,
        fc1_w.reshape(1, H), r(fc1_b), r(ln1_g), r(ln1_b),
        upd_w[:H], upd_w[H:], r(upd_b),
        fc2_w, r(fc2_b), r(ln2_g), r(ln2_b),
        out_w.reshape(1, H), out_b.reshape(1, 1), skip_w.reshape(1, 1),
    )
    return refined.reshape(B, L)
